# Initial kernel scaffold; baseline (speedup 1.0000x reference)
import jax
import jax.numpy as jnp
from jax.experimental import pallas as pl


def kernel(x, table):
    raise NotImplementedError("write your pallas kernel here")


# speedup vs baseline: 1.1113x; 1.1113x over previous
import functools

import jax
import jax.numpy as jnp
from jax import lax
from jax.experimental import pallas as pl
from jax.experimental.pallas import tpu as pltpu
from jax.experimental.pallas import tpu_sc as plsc

BATCH = 16384
HIST = 50
DIM = 32
TOTAL = BATCH * HIST

NUM_CORES = 2
NUM_SUBCORES = 16
NUM_WORKERS = NUM_CORES * NUM_SUBCORES
B_PER_W = TOTAL // NUM_WORKERS
CHUNK = 3200
NCHUNK = B_PER_W // CHUNK

_MESH = plsc.VectorSubcoreMesh(core_axis_name="c", subcore_axis_name="s")


@functools.partial(
    pl.kernel,
    out_type=jax.ShapeDtypeStruct((TOTAL, DIM), jnp.float32),
    mesh=_MESH,
    compiler_params=pltpu.CompilerParams(use_tc_tiling_on_sc=False),
    scratch_types=[
        pltpu.VMEM((CHUNK,), jnp.int32),
        pltpu.VMEM((CHUNK, DIM), jnp.float32),
        pltpu.SemaphoreType.DMA,
    ],
)
def _gather_kernel(x_hbm, table_hbm, out_hbm, idx_v, rows_v, sem):
    wid = lax.axis_index("s") * NUM_CORES + lax.axis_index("c")
    base = wid * B_PER_W

    def body(i, carry):
        off = base + i * CHUNK
        pltpu.sync_copy(x_hbm.at[pl.ds(off, CHUNK)], idx_v)
        pltpu.async_copy(table_hbm.at[idx_v], rows_v, sem).wait()
        pltpu.sync_copy(rows_v, out_hbm.at[pl.ds(off, CHUNK)])
        return carry

    lax.fori_loop(0, NCHUNK, body, 0)


def kernel(x, table):
    flat = _gather_kernel(x.reshape(TOTAL).astype(jnp.int32), table)
    return flat.reshape(BATCH, HIST, DIM)


# speedup vs baseline: 1.3485x; 1.2135x over previous
import functools

import jax
import jax.numpy as jnp
from jax import lax
from jax.experimental import pallas as pl
from jax.experimental.pallas import tpu as pltpu
from jax.experimental.pallas import tpu_sc as plsc

BATCH = 16384
HIST = 50
DIM = 32

NUM_CORES = 2
NUM_SUBCORES = 16
NUM_WORKERS = NUM_CORES * NUM_SUBCORES

BBLK = 512
NBLK = BATCH // BBLK
UNITS = HIST * NBLK
UNITS_PER_W = UNITS // NUM_WORKERS

_MESH = plsc.VectorSubcoreMesh(core_axis_name="c", subcore_axis_name="s")


@functools.partial(
    pl.kernel,
    out_type=jax.ShapeDtypeStruct((HIST, DIM, BATCH), jnp.float32),
    mesh=_MESH,
    compiler_params=pltpu.CompilerParams(needs_layout_passes=False),
    scratch_types=[
        pltpu.VMEM((BBLK,), jnp.int32),
        pltpu.VMEM((BBLK,), jnp.int32),
        pltpu.VMEM((BBLK, 128), jnp.float32),
        pltpu.VMEM((DIM, BBLK), jnp.float32),
        pltpu.SemaphoreType.DMA,
    ],
)
def _sc_lookup(xt_hbm, tbl_hbm, out_hbm, idx_v, jid_v, gat_v, tr_v, sem):
    wid = lax.axis_index("s") * NUM_CORES + lax.axis_index("c")
    lanes = lax.iota(jnp.int32, 16)

    def unit(u, carry):
        h = u // NBLK
        b0 = (u % NBLK) * BBLK
        pltpu.sync_copy(xt_hbm.at[h, pl.ds(b0, BBLK)], idx_v)

        def pack(g, c):
            v = idx_v[pl.ds(g * 16, 16)]
            jid_v[pl.ds(g * 16, 16)] = lax.shift_right_logical(v, 2)
            return c

        lax.fori_loop(0, BBLK // 16, pack, 0)
        pltpu.async_copy(tbl_hbm.at[jid_v], gat_v, sem).wait()

        def xpose(g, c):
            brow = lanes + g * 16
            sub = (idx_v[pl.ds(g * 16, 16)] & 3) * DIM
            for d in range(DIM):
                vals = plsc.load_gather(gat_v, [brow, sub + d])
                tr_v[d, pl.ds(g * 16, 16)] = vals
            return c

        lax.fori_loop(0, BBLK // 16, xpose, 0)
        pltpu.sync_copy(tr_v, out_hbm.at[h, :, pl.ds(b0, BBLK)])
        return carry

    lax.fori_loop(wid * UNITS_PER_W, (wid + 1) * UNITS_PER_W, unit, 0)


def kernel(x, table):
    xt = x.T
    tbl = table.reshape(250000, 128)
    ot = _sc_lookup(xt.astype(jnp.int32), tbl)
    return ot.transpose(2, 0, 1)


# speedup vs baseline: 1.3527x; 1.0031x over previous
import functools

import jax
import jax.numpy as jnp
from jax import lax
from jax.experimental import pallas as pl
from jax.experimental.pallas import tpu as pltpu
from jax.experimental.pallas import tpu_sc as plsc

BATCH = 16384
HIST = 50
DIM = 32

NUM_CORES = 2
NUM_SUBCORES = 16
NUM_WORKERS = NUM_CORES * NUM_SUBCORES

BBLK = 512
NBLK = BATCH // BBLK
UNITS = HIST * NBLK
UNITS_PER_W = UNITS // NUM_WORKERS

_MESH = plsc.VectorSubcoreMesh(core_axis_name="c", subcore_axis_name="s")


@functools.partial(
    pl.kernel,
    out_type=jax.ShapeDtypeStruct((HIST, DIM, BATCH), jnp.float32),
    mesh=_MESH,
    compiler_params=pltpu.CompilerParams(needs_layout_passes=False),
    scratch_types=[
        pltpu.VMEM((BBLK,), jnp.int32),
        pltpu.VMEM((BBLK,), jnp.int32),
        pltpu.VMEM((BBLK, 128), jnp.float32),
        pltpu.VMEM((DIM, BBLK), jnp.float32),
        pltpu.SemaphoreType.DMA,
    ],
)
def _sc_lookup(xt_hbm, tbl_hbm, out_hbm, idx_v, jid_v, gat_v, tr_v, sem):
    wid = lax.axis_index("s") * NUM_CORES + lax.axis_index("c")
    lanes = lax.iota(jnp.int32, 16)

    def unit(u, carry):
        h = u // NBLK
        b0 = (u % NBLK) * BBLK
        with jax.named_scope("idx_stage"):
            pltpu.sync_copy(xt_hbm.at[h, pl.ds(b0, BBLK)], idx_v)

            def pack(g, c):
                v = idx_v[pl.ds(g * 16, 16)]
                jid_v[pl.ds(g * 16, 16)] = lax.shift_right_logical(v, 2)
                return c

            lax.fori_loop(0, BBLK // 16, pack, 0)
        with jax.named_scope("row_gather"):
            pltpu.async_copy(tbl_hbm.at[jid_v], gat_v, sem).wait()

        with jax.named_scope("xpose"):
            def xpose(g, c):
                brow = lanes + g * 16
                sub = (idx_v[pl.ds(g * 16, 16)] & 3) * DIM
                for d in range(DIM):
                    vals = plsc.load_gather(gat_v, [brow, sub + d])
                    tr_v[d, pl.ds(g * 16, 16)] = vals
                return c

            lax.fori_loop(0, BBLK // 16, xpose, 0)
        with jax.named_scope("out_store"):
            pltpu.sync_copy(tr_v, out_hbm.at[h, :, pl.ds(b0, BBLK)])
        return carry

    lax.fori_loop(wid * UNITS_PER_W, (wid + 1) * UNITS_PER_W, unit, 0)


def kernel(x, table):
    xt = x.T
    tbl = table.reshape(250000, 128)
    ot = _sc_lookup(xt.astype(jnp.int32), tbl)
    return ot.transpose(2, 0, 1)


# speedup vs baseline: 1.9096x; 1.4117x over previous
import functools

import jax
import jax.numpy as jnp
from jax import lax
from jax.experimental import pallas as pl
from jax.experimental.pallas import tpu as pltpu
from jax.experimental.pallas import tpu_sc as plsc

BATCH = 16384
HIST = 50
DIM = 32

NUM_CORES = 2
NUM_SUBCORES = 16
NUM_WORKERS = NUM_CORES * NUM_SUBCORES

BBLK = 256
NBLK = BATCH // BBLK
UNITS = HIST * NBLK
UNITS_PER_W = UNITS // NUM_WORKERS
NGRP = BBLK // 16

_MESH = plsc.VectorSubcoreMesh(core_axis_name="c", subcore_axis_name="s")


@functools.partial(
    pl.kernel,
    out_type=jax.ShapeDtypeStruct((HIST, DIM, BATCH), jnp.float32),
    mesh=_MESH,
    compiler_params=pltpu.CompilerParams(needs_layout_passes=False),
    scratch_types=[
        pltpu.VMEM((BBLK,), jnp.int32),
        pltpu.VMEM((BBLK,), jnp.int32),
        pltpu.VMEM((BBLK,), jnp.int32),
        pltpu.VMEM((BBLK,), jnp.int32),
        pltpu.VMEM((BBLK, 128), jnp.float32),
        pltpu.VMEM((BBLK, 128), jnp.float32),
        pltpu.VMEM((DIM, BBLK), jnp.float32),
        pltpu.SemaphoreType.DMA,
        pltpu.SemaphoreType.DMA,
    ],
)
def _sc_lookup(xt_hbm, tbl_hbm, out_hbm, idx0, idx1, jid0, jid1, gat0, gat1,
               tr_v, sem0, sem1):
    wid = lax.axis_index("s") * NUM_CORES + lax.axis_index("c")
    base = wid * UNITS_PER_W
    lanes = lax.iota(jnp.int32, 16)
    slots = ((idx0, jid0, gat0, sem0), (idx1, jid1, gat1, sem1))

    def stage(u, buf):
        idx_v, jid_v, gat_v, sem = slots[buf]
        h = u // NBLK
        b0 = (u % NBLK) * BBLK
        pltpu.sync_copy(xt_hbm.at[h, pl.ds(b0, BBLK)], idx_v)

        @plsc.parallel_loop(0, NGRP)
        def pack(g):
            v = idx_v[pl.ds(g * 16, 16)]
            jid_v[pl.ds(g * 16, 16)] = lax.shift_right_logical(v, 2)

        pltpu.async_copy(tbl_hbm.at[jid_v], gat_v, sem)

    def drain(u, buf):
        idx_v, jid_v, gat_v, sem = slots[buf]
        h = u // NBLK
        b0 = (u % NBLK) * BBLK
        pltpu.make_async_copy(tbl_hbm.at[jid_v], gat_v, sem).wait()

        @plsc.parallel_loop(0, NGRP, unroll=2)
        def xpose(g):
            brow = lanes + g * 16
            sub = (idx_v[pl.ds(g * 16, 16)] & 3) * DIM
            for d in range(DIM):
                vals = plsc.load_gather(gat_v, [brow, sub + d])
                tr_v[d, pl.ds(g * 16, 16)] = vals

        pltpu.sync_copy(tr_v, out_hbm.at[h, :, pl.ds(b0, BBLK)])

    stage(base, 0)

    def pair(i, carry):
        u = base + 2 * i
        stage(u + 1, 1)
        drain(u, 0)

        @pl.when(2 * i + 2 < UNITS_PER_W)
        def _():
            stage(u + 2, 0)

        drain(u + 1, 1)
        return carry

    lax.fori_loop(0, UNITS_PER_W // 2, pair, 0)


def kernel(x, table):
    xt = x.T
    tbl = table.reshape(250000, 128)
    ot = _sc_lookup(xt.astype(jnp.int32), tbl)
    return ot.transpose(2, 0, 1)
